# baseline (speedup 1.0000x reference)
import functools

import jax
import jax.numpy as jnp
from jax import lax
from jax.experimental import pallas as pl
from jax.experimental.pallas import tpu as pltpu
from jax.experimental.pallas import tpu_sc as plsc

N = 10000
E = 320000
IN = 16
H = 16
EA = 4

NC = 2
NS = 16
NW = NC * NS
NCHK = 2
ECH = E // NCHK
PW = ECH // NW
CH = 40
NCHUNK = PW // CH
GRP = 5
NGRP = NCHUNK // GRP
ZROWS = N // NS


@functools.cache
def _mesh():
  return plsc.VectorSubcoreMesh(core_axis_name="c", subcore_axis_name="s")


_SC_PARAMS = pltpu.CompilerParams(use_tc_tiling_on_sc=False)


def _sc_gather(table, idx3):
  @functools.partial(
      pl.kernel,
      mesh=_mesh(),
      out_type=jax.ShapeDtypeStruct((ECH, 16), jnp.float32),
      scratch_types=[
          pltpu.VMEM((NCHUNK, CH), jnp.int32),
          pltpu.VMEM((GRP, CH, 16), jnp.float32),
          pltpu.SemaphoreType.DMA,
          pltpu.SemaphoreType.DMA,
      ],
      compiler_params=_SC_PARAMS,
  )
  def k(table_hbm, idx_hbm, out_hbm, idx_v, rows_v, gsem, ssem):
    wid = lax.axis_index("s") * NC + lax.axis_index("c")
    base = wid * PW
    pltpu.sync_copy(idx_hbm.at[wid], idx_v)

    def group(g, carry):
      j0 = g * GRP

      @pl.when(g > 0)
      def _drain_stores():
        for b in range(GRP):
          pltpu.make_async_copy(
              rows_v.at[b], out_hbm.at[pl.ds(base, CH)], ssem).wait()

      for b in range(GRP):
        pltpu.async_copy(table_hbm.at[idx_v.at[j0 + b]], rows_v.at[b], gsem)
      for b in range(GRP):
        pltpu.make_async_copy(
            table_hbm.at[idx_v.at[j0 + b]], rows_v.at[b], gsem).wait()
      for b in range(GRP):
        pltpu.async_copy(
            rows_v.at[b], out_hbm.at[pl.ds(base + (j0 + b) * CH, CH)], ssem)
      return carry

    lax.fori_loop(0, NGRP, group, 0)
    for b in range(GRP):
      pltpu.make_async_copy(
          rows_v.at[b], out_hbm.at[pl.ds(base, CH)], ssem).wait()

  return k(table, idx3)


def _sc_scatter(msg, idx3, with_cnt):
  outs = [jax.ShapeDtypeStruct((NC, N, 16), jnp.float32)]
  scratch = [
      pltpu.VMEM((NCHUNK, CH), jnp.int32),
      pltpu.VMEM((GRP, CH, 16), jnp.float32),
      pltpu.VMEM((ZROWS, 16), jnp.float32),
      pltpu.VMEM_SHARED((N, 16), jnp.float32),
      pltpu.SemaphoreType.DMA,
      pltpu.SemaphoreType.DMA,
      pltpu.SemaphoreType.DMA,
  ]
  if with_cnt:
    outs.append(jax.ShapeDtypeStruct((NC, N, 16), jnp.float32))
    scratch.insert(2, pltpu.VMEM((CH, 16), jnp.float32))
    scratch.insert(4, pltpu.VMEM_SHARED((N, 16), jnp.float32))

  @functools.partial(
      pl.kernel, mesh=_mesh(), out_type=outs, scratch_types=scratch,
      compiler_params=_SC_PARAMS)
  def k(*refs):
    if with_cnt:
      (msg_hbm, idx_hbm, out_sum, out_cnt,
       idx_v, rows_v, ones_v, stage_v, acc, cacc, lsem, asem, csem) = refs
    else:
      (msg_hbm, idx_hbm, out_sum,
       idx_v, rows_v, stage_v, acc, lsem, asem, csem) = refs
    sid = lax.axis_index("s")
    cid = lax.axis_index("c")
    wid = sid * NC + cid
    base = wid * PW

    def zbody(r, carry):
      stage_v[r, :] = jnp.zeros((16,), jnp.float32)
      return carry

    lax.fori_loop(0, ZROWS, zbody, 0)
    pltpu.sync_copy(stage_v, acc.at[pl.ds(sid * ZROWS, ZROWS)])
    if with_cnt:
      pltpu.sync_copy(stage_v, cacc.at[pl.ds(sid * ZROWS, ZROWS)])

      def obody(r, carry):
        ones_v[r, :] = jnp.ones((16,), jnp.float32)
        return carry

      lax.fori_loop(0, CH, obody, 0)
    pltpu.sync_copy(idx_hbm.at[wid], idx_v)
    plsc.subcore_barrier()

    def group(g, carry):
      j0 = g * GRP

      @pl.when(g > 0)
      def _drain_adds():
        for b in range(GRP):
          pltpu.make_async_copy(
              rows_v.at[b], acc.at[idx_v.at[j0 + b]], asem).wait()
          if with_cnt:
            pltpu.make_async_copy(
                ones_v, cacc.at[idx_v.at[j0 + b]], csem).wait()

      for b in range(GRP):
        pltpu.async_copy(
            msg_hbm.at[pl.ds(base + (j0 + b) * CH, CH)], rows_v.at[b], lsem)
      for b in range(GRP):
        pltpu.make_async_copy(
            msg_hbm.at[pl.ds(base, CH)], rows_v.at[b], lsem).wait()
      for b in range(GRP):
        pltpu.async_copy(rows_v.at[b], acc.at[idx_v.at[j0 + b]], asem,
                         add=True)
        if with_cnt:
          pltpu.async_copy(ones_v, cacc.at[idx_v.at[j0 + b]], csem, add=True)
      return carry

    lax.fori_loop(0, NGRP, group, 0)
    for b in range(GRP):
      pltpu.make_async_copy(rows_v.at[b], acc.at[idx_v.at[b]], asem).wait()
      if with_cnt:
        pltpu.make_async_copy(ones_v, cacc.at[idx_v.at[b]], csem).wait()
    plsc.subcore_barrier()

    pltpu.sync_copy(acc.at[pl.ds(sid * ZROWS, ZROWS)], stage_v)
    pltpu.sync_copy(stage_v, out_sum.at[cid, pl.ds(sid * ZROWS, ZROWS)])
    if with_cnt:
      pltpu.sync_copy(cacc.at[pl.ds(sid * ZROWS, ZROWS)], stage_v)
      pltpu.sync_copy(stage_v, out_cnt.at[cid, pl.ds(sid * ZROWS, ZROWS)])

  res = k(msg, idx3)
  return res if with_cnt else res[0]


_TB = 6400


def _tc_edge_msgs(eap, xjp, Wa_bd, ba8, Wb_bd, bb8, R_bd, S_bd):
  TB8 = _TB // 8

  def body(eap_ref, xj_ref, wa, ba_r, wb, bb_r, r_r, s_r, out_ref):
    hp = jnp.maximum(
        jnp.dot(eap_ref[...], wa[...], preferred_element_type=jnp.float32)
        + ba_r[...], 0.0)
    hp = hp.astype(jnp.bfloat16)
    wp = jnp.dot(hp, wb[...],
                 preferred_element_type=jnp.float32) + bb_r[...]
    xep = jnp.dot(xj_ref[...].astype(jnp.bfloat16), r_r[...],
                  preferred_element_type=jnp.float32)
    out_ref[...] = jnp.dot(
        (wp * xep).astype(jnp.bfloat16), s_r[...],
        preferred_element_type=jnp.float32)

  zero = lambda i: (0, 0)
  return pl.pallas_call(
      body,
      grid=(ECH // _TB,),
      in_specs=[
          pl.BlockSpec((TB8, 8 * EA), lambda i: (i, 0)),
          pl.BlockSpec((TB8, 128), lambda i: (i, 0)),
          pl.BlockSpec((8 * EA, 8 * 32), zero),
          pl.BlockSpec((1, 8 * 32), zero),
          pl.BlockSpec((8 * 32, 8 * IN * H), zero),
          pl.BlockSpec((1, 8 * IN * H), zero),
          pl.BlockSpec((128, 8 * IN * H), zero),
          pl.BlockSpec((8 * IN * H, 128), zero),
      ],
      out_specs=pl.BlockSpec((TB8, 128), lambda i: (i, 0)),
      out_shape=jax.ShapeDtypeStruct((ECH // 8, 128), jnp.float32),
  )(eap, xjp, Wa_bd, ba8.reshape(1, 8 * 32), Wb_bd,
    bb8.reshape(1, 8 * IN * H), R_bd, S_bd)


NP8 = N // 8


def _tc_finalize1(sa, sb, ca, cb, x_p, root_bd, bias_t):
  def body(sa_r, sb_r, ca_r, cb_r, x_ref, r_ref, b_ref, h_ref, rinv_ref):
    cnt = ca_r[0] + ca_r[1] + cb_r[0] + cb_r[1]
    rinv = 1.0 / jnp.maximum(cnt, 1.0)
    mean = (sa_r[0] + sa_r[1] + sb_r[0] + sb_r[1]) * rinv
    h = mean + jnp.dot(
        x_ref[...], r_ref[...], preferred_element_type=jnp.float32) + b_ref[...]
    h_ref[...] = jnp.maximum(h, 0.0)
    rinv_ref[...] = rinv

  return pl.pallas_call(
      body,
      out_shape=[
          jax.ShapeDtypeStruct((NP8, 128), jnp.float32),
          jax.ShapeDtypeStruct((NP8, 128), jnp.float32),
      ],
  )(sa, sb, ca, cb, x_p, root_bd, bias_t.reshape(1, 128))


def _tc_finalize2(sa, sb, rinv_p, h1_p, root_bd, bias_t, Wl_bd, bl_t):
  def body(sa_r, sb_r, rinv_ref, h1_ref, r_ref, b_ref, wl_ref, bl_ref,
           out_ref):
    mean = (sa_r[0] + sa_r[1] + sb_r[0] + sb_r[1]) * rinv_ref[...]
    h2 = mean + jnp.dot(
        h1_ref[...], r_ref[...], preferred_element_type=jnp.float32) + b_ref[...]
    h2 = jnp.maximum(h2, 0.0)
    out_ref[...] = jnp.dot(
        h2, wl_ref[...], preferred_element_type=jnp.float32) + bl_ref[...]

  return pl.pallas_call(
      body,
      out_shape=jax.ShapeDtypeStruct((NP8, 8), jnp.float32),
  )(sa, sb, rinv_p, h1_p, root_bd, bias_t.reshape(1, 128), Wl_bd,
    bl_t.reshape(1, 8))


def kernel(x, edge_index, edge_attr, W1a, b1a, W1b, b1b, root1, bias1,
           W2a, b2a, W2b, b2b, root2, bias2, Wl, bl):
  src4 = edge_index[0].astype(jnp.int32).reshape(NCHK, NW, NCHUNK, CH)
  dst4 = edge_index[1].astype(jnp.int32).reshape(NCHK, NW, NCHUNK, CH)

  c = jnp.arange(IN * H, dtype=jnp.int32)
  R = (jnp.arange(IN, dtype=jnp.int32)[:, None] == (c // H)[None, :]
       ).astype(jnp.float32)
  S = ((c % H)[:, None] == jnp.arange(H, dtype=jnp.int32)[None, :]
       ).astype(jnp.float32)

  eye8 = jnp.eye(8, dtype=jnp.float32)
  bf = jnp.bfloat16
  W1a_bd = jnp.kron(eye8, W1a).astype(bf)
  W1b_bd = jnp.kron(eye8, W1b).astype(bf)
  W2a_bd = jnp.kron(eye8, W2a).astype(bf)
  W2b_bd = jnp.kron(eye8, W2b).astype(bf)
  R_bd = jnp.kron(eye8, R).astype(bf)
  S_bd = jnp.kron(eye8, S).astype(bf)
  b1a8 = jnp.tile(b1a, 8)
  b1b8 = jnp.tile(b1b, 8)
  b2a8 = jnp.tile(b2a, 8)
  b2b8 = jnp.tile(b2b, 8)
  root1_bd = jnp.kron(eye8, root1)
  root2_bd = jnp.kron(eye8, root2)
  Wl_bd = jnp.kron(eye8, Wl)
  bias1_t = jnp.tile(bias1, 8)
  bias2_t = jnp.tile(bias2, 8)
  bl_t = jnp.tile(bl, 8)
  eap = edge_attr.astype(bf).reshape(NCHK, ECH // 8, 8 * EA)

  def layer(table, Wa_bd, ba8, Wb_bd, bb8, with_cnt):
    sums = []
    cnts = []
    for ci in range(NCHK):
      xj = _sc_gather(table, src4[ci])
      msg = _tc_edge_msgs(eap[ci], xj.reshape(ECH // 8, 128),
                          Wa_bd, ba8, Wb_bd, bb8, R_bd, S_bd)
      res = _sc_scatter(msg.reshape(ECH, 16), dst4[ci], with_cnt)
      if with_cnt:
        sums.append(res[0].reshape(NC, NP8, 128))
        cnts.append(res[1].reshape(NC, NP8, 128))
      else:
        sums.append(res.reshape(NC, NP8, 128))
    return sums, cnts

  sums1, cnts1 = layer(x, W1a_bd, b1a8, W1b_bd, b1b8, True)
  h1p, rinvp = _tc_finalize1(sums1[0], sums1[1], cnts1[0], cnts1[1],
                             x.reshape(NP8, 128), root1_bd, bias1_t)

  sums2, _ = layer(h1p.reshape(N, H), W2a_bd, b2a8, W2b_bd, b2b8, False)
  out = _tc_finalize2(sums2[0], sums2[1], rinvp, h1p,
                      root2_bd, bias2_t, Wl_bd, bl_t)
  return out.reshape(N)


# speedup vs baseline: 1.1085x; 1.1085x over previous
import functools

import jax
import jax.numpy as jnp
from jax import lax
from jax.experimental import pallas as pl
from jax.experimental.pallas import tpu as pltpu
from jax.experimental.pallas import tpu_sc as plsc

N = 10000
E = 320000
IN = 16
H = 16
EA = 4

NC = 2
NS = 16
NW = NC * NS
NCHK = 1
ECH = E // NCHK
PW = ECH // NW
CH = 80
NCHUNK = PW // CH
GRP = 5
NGRP = NCHUNK // GRP
ZROWS = N // NS


@functools.cache
def _mesh():
  return plsc.VectorSubcoreMesh(core_axis_name="c", subcore_axis_name="s")


_SC_PARAMS = pltpu.CompilerParams(use_tc_tiling_on_sc=False)


def _sc_gather(table, idx3):
  @functools.partial(
      pl.kernel,
      mesh=_mesh(),
      out_type=jax.ShapeDtypeStruct((ECH, 16), jnp.float32),
      scratch_types=[
          pltpu.VMEM((NCHUNK, CH), jnp.int32),
          pltpu.VMEM((GRP, CH, 16), jnp.float32),
          pltpu.SemaphoreType.DMA,
          pltpu.SemaphoreType.DMA,
      ],
      compiler_params=_SC_PARAMS,
  )
  def k(table_hbm, idx_hbm, out_hbm, idx_v, rows_v, gsem, ssem):
    wid = lax.axis_index("s") * NC + lax.axis_index("c")
    base = wid * PW
    pltpu.sync_copy(idx_hbm.at[wid], idx_v)

    def group(g, carry):
      j0 = g * GRP

      @pl.when(g > 0)
      def _drain_stores():
        for b in range(GRP):
          pltpu.make_async_copy(
              rows_v.at[b], out_hbm.at[pl.ds(base, CH)], ssem).wait()

      for b in range(GRP):
        pltpu.async_copy(table_hbm.at[idx_v.at[j0 + b]], rows_v.at[b], gsem)
      for b in range(GRP):
        pltpu.make_async_copy(
            table_hbm.at[idx_v.at[j0 + b]], rows_v.at[b], gsem).wait()
      for b in range(GRP):
        pltpu.async_copy(
            rows_v.at[b], out_hbm.at[pl.ds(base + (j0 + b) * CH, CH)], ssem)
      return carry

    lax.fori_loop(0, NGRP, group, 0)
    for b in range(GRP):
      pltpu.make_async_copy(
          rows_v.at[b], out_hbm.at[pl.ds(base, CH)], ssem).wait()

  return k(table, idx3)


def _sc_scatter(msg, idx3, with_cnt):
  outs = [jax.ShapeDtypeStruct((NC, N, 16), jnp.float32)]
  scratch = [
      pltpu.VMEM((NCHUNK, CH), jnp.int32),
      pltpu.VMEM((GRP, CH, 16), jnp.float32),
      pltpu.VMEM((ZROWS, 16), jnp.float32),
      pltpu.VMEM_SHARED((N, 16), jnp.float32),
      pltpu.SemaphoreType.DMA,
      pltpu.SemaphoreType.DMA,
      pltpu.SemaphoreType.DMA,
  ]
  if with_cnt:
    outs.append(jax.ShapeDtypeStruct((NC, N, 16), jnp.float32))
    scratch.insert(2, pltpu.VMEM((CH, 16), jnp.float32))
    scratch.insert(4, pltpu.VMEM_SHARED((N, 16), jnp.float32))

  @functools.partial(
      pl.kernel, mesh=_mesh(), out_type=outs, scratch_types=scratch,
      compiler_params=_SC_PARAMS)
  def k(*refs):
    if with_cnt:
      (msg_hbm, idx_hbm, out_sum, out_cnt,
       idx_v, rows_v, ones_v, stage_v, acc, cacc, lsem, asem, csem) = refs
    else:
      (msg_hbm, idx_hbm, out_sum,
       idx_v, rows_v, stage_v, acc, lsem, asem, csem) = refs
    sid = lax.axis_index("s")
    cid = lax.axis_index("c")
    wid = sid * NC + cid
    base = wid * PW

    def zbody(r, carry):
      stage_v[r, :] = jnp.zeros((16,), jnp.float32)
      return carry

    lax.fori_loop(0, ZROWS, zbody, 0)
    pltpu.sync_copy(stage_v, acc.at[pl.ds(sid * ZROWS, ZROWS)])
    if with_cnt:
      pltpu.sync_copy(stage_v, cacc.at[pl.ds(sid * ZROWS, ZROWS)])

      def obody(r, carry):
        ones_v[r, :] = jnp.ones((16,), jnp.float32)
        return carry

      lax.fori_loop(0, CH, obody, 0)
    pltpu.sync_copy(idx_hbm.at[wid], idx_v)
    plsc.subcore_barrier()

    def group(g, carry):
      j0 = g * GRP

      @pl.when(g > 0)
      def _drain_adds():
        for b in range(GRP):
          pltpu.make_async_copy(
              rows_v.at[b], acc.at[idx_v.at[j0 + b]], asem).wait()
          if with_cnt:
            pltpu.make_async_copy(
                ones_v, cacc.at[idx_v.at[j0 + b]], csem).wait()

      for b in range(GRP):
        pltpu.async_copy(
            msg_hbm.at[pl.ds(base + (j0 + b) * CH, CH)], rows_v.at[b], lsem)
      for b in range(GRP):
        pltpu.make_async_copy(
            msg_hbm.at[pl.ds(base, CH)], rows_v.at[b], lsem).wait()
      for b in range(GRP):
        pltpu.async_copy(rows_v.at[b], acc.at[idx_v.at[j0 + b]], asem,
                         add=True)
        if with_cnt:
          pltpu.async_copy(ones_v, cacc.at[idx_v.at[j0 + b]], csem, add=True)
      return carry

    lax.fori_loop(0, NGRP, group, 0)
    for b in range(GRP):
      pltpu.make_async_copy(rows_v.at[b], acc.at[idx_v.at[b]], asem).wait()
      if with_cnt:
        pltpu.make_async_copy(ones_v, cacc.at[idx_v.at[b]], csem).wait()
    plsc.subcore_barrier()

    pltpu.sync_copy(acc.at[pl.ds(sid * ZROWS, ZROWS)], stage_v)
    pltpu.sync_copy(stage_v, out_sum.at[cid, pl.ds(sid * ZROWS, ZROWS)])
    if with_cnt:
      pltpu.sync_copy(cacc.at[pl.ds(sid * ZROWS, ZROWS)], stage_v)
      pltpu.sync_copy(stage_v, out_cnt.at[cid, pl.ds(sid * ZROWS, ZROWS)])

  res = k(msg, idx3)
  return res if with_cnt else res[0]


_TB = 6400


def _tc_edge_msgs(eap, xjp, Wa_bd, ba8, Wb_bd, bb8, R_bd, S_bd):
  TB8 = _TB // 8

  def body(eap_ref, xj_ref, wa, ba_r, wb, bb_r, r_r, s_r, out_ref):
    hp = jnp.maximum(
        jnp.dot(eap_ref[...], wa[...], preferred_element_type=jnp.float32)
        + ba_r[...], 0.0)
    hp = hp.astype(jnp.bfloat16)
    wp = jnp.dot(hp, wb[...],
                 preferred_element_type=jnp.float32) + bb_r[...]
    xep = jnp.dot(xj_ref[...].astype(jnp.bfloat16), r_r[...],
                  preferred_element_type=jnp.float32)
    out_ref[...] = jnp.dot(
        (wp * xep).astype(jnp.bfloat16), s_r[...],
        preferred_element_type=jnp.float32)

  zero = lambda i: (0, 0)
  return pl.pallas_call(
      body,
      grid=(ECH // _TB,),
      in_specs=[
          pl.BlockSpec((TB8, 8 * EA), lambda i: (i, 0)),
          pl.BlockSpec((TB8, 128), lambda i: (i, 0)),
          pl.BlockSpec((8 * EA, 8 * 32), zero),
          pl.BlockSpec((1, 8 * 32), zero),
          pl.BlockSpec((8 * 32, 8 * IN * H), zero),
          pl.BlockSpec((1, 8 * IN * H), zero),
          pl.BlockSpec((128, 8 * IN * H), zero),
          pl.BlockSpec((8 * IN * H, 128), zero),
      ],
      out_specs=pl.BlockSpec((TB8, 128), lambda i: (i, 0)),
      out_shape=jax.ShapeDtypeStruct((ECH // 8, 128), jnp.float32),
  )(eap, xjp, Wa_bd, ba8.reshape(1, 8 * 32), Wb_bd,
    bb8.reshape(1, 8 * IN * H), R_bd, S_bd)


NP8 = N // 8


def _tc_finalize1(sums_l, cnts_l, x_p, root_bd, bias_t):
  ns = len(sums_l)

  def body(*refs):
    s_refs = refs[:ns]
    c_refs = refs[ns:2 * ns]
    x_ref, r_ref, b_ref, h_ref, rinv_ref = refs[2 * ns:]
    cnt = sum(c[0] + c[1] for c in c_refs)
    rinv = 1.0 / jnp.maximum(cnt, 1.0)
    mean = sum(s[0] + s[1] for s in s_refs) * rinv
    h = mean + jnp.dot(
        x_ref[...], r_ref[...], preferred_element_type=jnp.float32) + b_ref[...]
    h_ref[...] = jnp.maximum(h, 0.0)
    rinv_ref[...] = rinv

  return pl.pallas_call(
      body,
      out_shape=[
          jax.ShapeDtypeStruct((NP8, 128), jnp.float32),
          jax.ShapeDtypeStruct((NP8, 128), jnp.float32),
      ],
  )(*sums_l, *cnts_l, x_p, root_bd, bias_t.reshape(1, 128))


def _tc_finalize2(sums_l, rinv_p, h1_p, root_bd, bias_t, Wl_bd, bl_t):
  ns = len(sums_l)

  def body(*refs):
    s_refs = refs[:ns]
    rinv_ref, h1_ref, r_ref, b_ref, wl_ref, bl_ref, out_ref = refs[ns:]
    mean = sum(s[0] + s[1] for s in s_refs) * rinv_ref[...]
    h2 = mean + jnp.dot(
        h1_ref[...], r_ref[...], preferred_element_type=jnp.float32) + b_ref[...]
    h2 = jnp.maximum(h2, 0.0)
    out_ref[...] = jnp.dot(
        h2, wl_ref[...], preferred_element_type=jnp.float32) + bl_ref[...]

  return pl.pallas_call(
      body,
      out_shape=jax.ShapeDtypeStruct((NP8, 8), jnp.float32),
  )(*sums_l, rinv_p, h1_p, root_bd, bias_t.reshape(1, 128), Wl_bd,
    bl_t.reshape(1, 8))


def kernel(x, edge_index, edge_attr, W1a, b1a, W1b, b1b, root1, bias1,
           W2a, b2a, W2b, b2b, root2, bias2, Wl, bl):
  src4 = edge_index[0].astype(jnp.int32).reshape(NCHK, NW, NCHUNK, CH)
  dst4 = edge_index[1].astype(jnp.int32).reshape(NCHK, NW, NCHUNK, CH)

  c = jnp.arange(IN * H, dtype=jnp.int32)
  R = (jnp.arange(IN, dtype=jnp.int32)[:, None] == (c // H)[None, :]
       ).astype(jnp.float32)
  S = ((c % H)[:, None] == jnp.arange(H, dtype=jnp.int32)[None, :]
       ).astype(jnp.float32)

  eye8 = jnp.eye(8, dtype=jnp.float32)
  bf = jnp.bfloat16
  W1a_bd = jnp.kron(eye8, W1a).astype(bf)
  W1b_bd = jnp.kron(eye8, W1b).astype(bf)
  W2a_bd = jnp.kron(eye8, W2a).astype(bf)
  W2b_bd = jnp.kron(eye8, W2b).astype(bf)
  R_bd = jnp.kron(eye8, R).astype(bf)
  S_bd = jnp.kron(eye8, S).astype(bf)
  b1a8 = jnp.tile(b1a, 8)
  b1b8 = jnp.tile(b1b, 8)
  b2a8 = jnp.tile(b2a, 8)
  b2b8 = jnp.tile(b2b, 8)
  root1_bd = jnp.kron(eye8, root1)
  root2_bd = jnp.kron(eye8, root2)
  Wl_bd = jnp.kron(eye8, Wl)
  bias1_t = jnp.tile(bias1, 8)
  bias2_t = jnp.tile(bias2, 8)
  bl_t = jnp.tile(bl, 8)
  eap = edge_attr.astype(bf).reshape(NCHK, ECH // 8, 8 * EA)

  def layer(table, Wa_bd, ba8, Wb_bd, bb8, with_cnt):
    sums = []
    cnts = []
    for ci in range(NCHK):
      xj = _sc_gather(table, src4[ci])
      msg = _tc_edge_msgs(eap[ci], xj.reshape(ECH // 8, 128),
                          Wa_bd, ba8, Wb_bd, bb8, R_bd, S_bd)
      res = _sc_scatter(msg.reshape(ECH, 16), dst4[ci], with_cnt)
      if with_cnt:
        sums.append(res[0].reshape(NC, NP8, 128))
        cnts.append(res[1].reshape(NC, NP8, 128))
      else:
        sums.append(res.reshape(NC, NP8, 128))
    return sums, cnts

  sums1, cnts1 = layer(x, W1a_bd, b1a8, W1b_bd, b1b8, True)
  h1p, rinvp = _tc_finalize1(sums1, cnts1, x.reshape(NP8, 128),
                             root1_bd, bias1_t)

  sums2, _ = layer(h1p.reshape(N, H), W2a_bd, b2a8, W2b_bd, b2b8, False)
  out = _tc_finalize2(sums2, rinvp, h1p, root2_bd, bias2_t, Wl_bd, bl_t)
  return out.reshape(N)


# speedup vs baseline: 1.1240x; 1.0140x over previous
import functools

import jax
import jax.numpy as jnp
from jax import lax
from jax.experimental import pallas as pl
from jax.experimental.pallas import tpu as pltpu
from jax.experimental.pallas import tpu_sc as plsc

N = 10000
E = 320000
IN = 16
H = 16
EA = 4

NC = 2
NS = 16
NW = NC * NS
NCHK = 1
ECH = E // NCHK
PW = ECH // NW
CH = 80
NCHUNK = PW // CH
GRP = 5
NGRP = NCHUNK // GRP
ZROWS = N // NS


@functools.cache
def _mesh():
  return plsc.VectorSubcoreMesh(core_axis_name="c", subcore_axis_name="s")


_SC_PARAMS = pltpu.CompilerParams(use_tc_tiling_on_sc=False)


def _sc_gather(table, idx3):
  @functools.partial(
      pl.kernel,
      mesh=_mesh(),
      out_type=jax.ShapeDtypeStruct((ECH, 16), jnp.float32),
      scratch_types=[
          pltpu.VMEM((NCHUNK, CH), jnp.int32),
          pltpu.VMEM((2 * GRP, CH, 16), jnp.float32),
          pltpu.SemaphoreType.DMA,
          pltpu.SemaphoreType.DMA,
      ],
      compiler_params=_SC_PARAMS,
  )
  def k(table_hbm, idx_hbm, out_hbm, idx_v, rows_v, gsem, ssem):
    wid = lax.axis_index("s") * NC + lax.axis_index("c")
    base = wid * PW
    pltpu.sync_copy(idx_hbm.at[wid], idx_v)

    def fire(g, p):
      j0 = g * GRP
      for b in range(GRP):
        pltpu.async_copy(
            table_hbm.at[idx_v.at[j0 + b]], rows_v.at[p * GRP + b], gsem)

    def retire(g, p):
      j0 = g * GRP
      for b in range(GRP):
        pltpu.make_async_copy(
            table_hbm.at[idx_v.at[j0 + b]], rows_v.at[p * GRP + b],
            gsem).wait()
      for b in range(GRP):
        pltpu.async_copy(
            rows_v.at[p * GRP + b],
            out_hbm.at[pl.ds(base + (j0 + b) * CH, CH)], ssem)

    def drain_stores(p):
      for b in range(GRP):
        pltpu.make_async_copy(
            rows_v.at[p * GRP + b], out_hbm.at[pl.ds(base, CH)], ssem).wait()

    fire(0, 0)

    def pair(q, carry):
      g = 2 * q

      @pl.when(g + 1 < NGRP)
      def _f1():
        fire(g + 1, 1)

      retire(g, 0)

      @pl.when(g + 2 < NGRP)
      def _f0():
        drain_stores(0)
        fire(g + 2, 0)

      @pl.when(g + 1 < NGRP)
      def _r1():
        retire(g + 1, 1)
        drain_stores(1)

      return carry

    lax.fori_loop(0, (NGRP + 1) // 2, pair, 0)
    drain_stores(0)

  return k(table, idx3)


def _sc_scatter(msg, idx3, with_cnt):
  outs = [jax.ShapeDtypeStruct((NC, N, 16), jnp.float32)]
  scratch = [
      pltpu.VMEM((NCHUNK, CH), jnp.int32),
      pltpu.VMEM((GRP, CH, 16), jnp.float32),
      pltpu.VMEM((ZROWS, 16), jnp.float32),
      pltpu.VMEM_SHARED((N, 16), jnp.float32),
      pltpu.SemaphoreType.DMA,
      pltpu.SemaphoreType.DMA,
      pltpu.SemaphoreType.DMA,
  ]
  if with_cnt:
    outs.append(jax.ShapeDtypeStruct((NC, N, 16), jnp.float32))
    scratch.insert(2, pltpu.VMEM((CH, 16), jnp.float32))
    scratch.insert(4, pltpu.VMEM_SHARED((N, 16), jnp.float32))

  @functools.partial(
      pl.kernel, mesh=_mesh(), out_type=outs, scratch_types=scratch,
      compiler_params=_SC_PARAMS)
  def k(*refs):
    if with_cnt:
      (msg_hbm, idx_hbm, out_sum, out_cnt,
       idx_v, rows_v, ones_v, stage_v, acc, cacc, lsem, asem, csem) = refs
    else:
      (msg_hbm, idx_hbm, out_sum,
       idx_v, rows_v, stage_v, acc, lsem, asem, csem) = refs
    sid = lax.axis_index("s")
    cid = lax.axis_index("c")
    wid = sid * NC + cid
    base = wid * PW

    def zbody(r, carry):
      stage_v[r, :] = jnp.zeros((16,), jnp.float32)
      return carry

    lax.fori_loop(0, ZROWS, zbody, 0)
    pltpu.sync_copy(stage_v, acc.at[pl.ds(sid * ZROWS, ZROWS)])
    if with_cnt:
      pltpu.sync_copy(stage_v, cacc.at[pl.ds(sid * ZROWS, ZROWS)])

      def obody(r, carry):
        ones_v[r, :] = jnp.ones((16,), jnp.float32)
        return carry

      lax.fori_loop(0, CH, obody, 0)
    pltpu.sync_copy(idx_hbm.at[wid], idx_v)
    plsc.subcore_barrier()

    def group(g, carry):
      j0 = g * GRP

      @pl.when(g > 0)
      def _drain_adds():
        for b in range(GRP):
          pltpu.make_async_copy(
              rows_v.at[b], acc.at[idx_v.at[j0 + b]], asem).wait()
          if with_cnt:
            pltpu.make_async_copy(
                ones_v, cacc.at[idx_v.at[j0 + b]], csem).wait()

      for b in range(GRP):
        pltpu.async_copy(
            msg_hbm.at[pl.ds(base + (j0 + b) * CH, CH)], rows_v.at[b], lsem)
      for b in range(GRP):
        pltpu.make_async_copy(
            msg_hbm.at[pl.ds(base, CH)], rows_v.at[b], lsem).wait()
      for b in range(GRP):
        pltpu.async_copy(rows_v.at[b], acc.at[idx_v.at[j0 + b]], asem,
                         add=True)
        if with_cnt:
          pltpu.async_copy(ones_v, cacc.at[idx_v.at[j0 + b]], csem, add=True)
      return carry

    lax.fori_loop(0, NGRP, group, 0)
    for b in range(GRP):
      pltpu.make_async_copy(rows_v.at[b], acc.at[idx_v.at[b]], asem).wait()
      if with_cnt:
        pltpu.make_async_copy(ones_v, cacc.at[idx_v.at[b]], csem).wait()
    plsc.subcore_barrier()

    pltpu.sync_copy(acc.at[pl.ds(sid * ZROWS, ZROWS)], stage_v)
    pltpu.sync_copy(stage_v, out_sum.at[cid, pl.ds(sid * ZROWS, ZROWS)])
    if with_cnt:
      pltpu.sync_copy(cacc.at[pl.ds(sid * ZROWS, ZROWS)], stage_v)
      pltpu.sync_copy(stage_v, out_cnt.at[cid, pl.ds(sid * ZROWS, ZROWS)])

  res = k(msg, idx3)
  return res if with_cnt else res[0]


_TB = 6400


def _tc_edge_msgs(eap, xjp, Wa_bd, ba8, Wb_bd, bb8, R_bd, S_bd):
  TB8 = _TB // 8

  def body(eap_ref, xj_ref, wa, ba_r, wb, bb_r, r_r, s_r, out_ref):
    hp = jnp.maximum(
        jnp.dot(eap_ref[...], wa[...], preferred_element_type=jnp.float32)
        + ba_r[...], 0.0)
    hp = hp.astype(jnp.bfloat16)
    wp = jnp.dot(hp, wb[...],
                 preferred_element_type=jnp.float32) + bb_r[...]
    xep = jnp.dot(xj_ref[...].astype(jnp.bfloat16), r_r[...],
                  preferred_element_type=jnp.float32)
    out_ref[...] = jnp.dot(
        (wp * xep).astype(jnp.bfloat16), s_r[...],
        preferred_element_type=jnp.float32)

  zero = lambda i: (0, 0)
  return pl.pallas_call(
      body,
      grid=(ECH // _TB,),
      in_specs=[
          pl.BlockSpec((TB8, 8 * EA), lambda i: (i, 0)),
          pl.BlockSpec((TB8, 128), lambda i: (i, 0)),
          pl.BlockSpec((8 * EA, 8 * 32), zero),
          pl.BlockSpec((1, 8 * 32), zero),
          pl.BlockSpec((8 * 32, 8 * IN * H), zero),
          pl.BlockSpec((1, 8 * IN * H), zero),
          pl.BlockSpec((128, 8 * IN * H), zero),
          pl.BlockSpec((8 * IN * H, 128), zero),
      ],
      out_specs=pl.BlockSpec((TB8, 128), lambda i: (i, 0)),
      out_shape=jax.ShapeDtypeStruct((ECH // 8, 128), jnp.float32),
  )(eap, xjp, Wa_bd, ba8.reshape(1, 8 * 32), Wb_bd,
    bb8.reshape(1, 8 * IN * H), R_bd, S_bd)


NP8 = N // 8


def _tc_finalize1(sums_l, cnts_l, x_p, root_bd, bias_t):
  ns = len(sums_l)

  def body(*refs):
    s_refs = refs[:ns]
    c_refs = refs[ns:2 * ns]
    x_ref, r_ref, b_ref, h_ref, rinv_ref = refs[2 * ns:]
    cnt = sum(c[0] + c[1] for c in c_refs)
    rinv = 1.0 / jnp.maximum(cnt, 1.0)
    mean = sum(s[0] + s[1] for s in s_refs) * rinv
    h = mean + jnp.dot(
        x_ref[...], r_ref[...], preferred_element_type=jnp.float32) + b_ref[...]
    h_ref[...] = jnp.maximum(h, 0.0)
    rinv_ref[...] = rinv

  return pl.pallas_call(
      body,
      out_shape=[
          jax.ShapeDtypeStruct((NP8, 128), jnp.float32),
          jax.ShapeDtypeStruct((NP8, 128), jnp.float32),
      ],
  )(*sums_l, *cnts_l, x_p, root_bd, bias_t.reshape(1, 128))


def _tc_finalize2(sums_l, rinv_p, h1_p, root_bd, bias_t, Wl_bd, bl_t):
  ns = len(sums_l)

  def body(*refs):
    s_refs = refs[:ns]
    rinv_ref, h1_ref, r_ref, b_ref, wl_ref, bl_ref, out_ref = refs[ns:]
    mean = sum(s[0] + s[1] for s in s_refs) * rinv_ref[...]
    h2 = mean + jnp.dot(
        h1_ref[...], r_ref[...], preferred_element_type=jnp.float32) + b_ref[...]
    h2 = jnp.maximum(h2, 0.0)
    out_ref[...] = jnp.dot(
        h2, wl_ref[...], preferred_element_type=jnp.float32) + bl_ref[...]

  return pl.pallas_call(
      body,
      out_shape=jax.ShapeDtypeStruct((NP8, 8), jnp.float32),
  )(*sums_l, rinv_p, h1_p, root_bd, bias_t.reshape(1, 128), Wl_bd,
    bl_t.reshape(1, 8))


def kernel(x, edge_index, edge_attr, W1a, b1a, W1b, b1b, root1, bias1,
           W2a, b2a, W2b, b2b, root2, bias2, Wl, bl):
  src4 = edge_index[0].astype(jnp.int32).reshape(NCHK, NW, NCHUNK, CH)
  dst4 = edge_index[1].astype(jnp.int32).reshape(NCHK, NW, NCHUNK, CH)

  c = jnp.arange(IN * H, dtype=jnp.int32)
  R = (jnp.arange(IN, dtype=jnp.int32)[:, None] == (c // H)[None, :]
       ).astype(jnp.float32)
  S = ((c % H)[:, None] == jnp.arange(H, dtype=jnp.int32)[None, :]
       ).astype(jnp.float32)

  eye8 = jnp.eye(8, dtype=jnp.float32)
  bf = jnp.bfloat16
  W1a_bd = jnp.kron(eye8, W1a).astype(bf)
  W1b_bd = jnp.kron(eye8, W1b).astype(bf)
  W2a_bd = jnp.kron(eye8, W2a).astype(bf)
  W2b_bd = jnp.kron(eye8, W2b).astype(bf)
  R_bd = jnp.kron(eye8, R).astype(bf)
  S_bd = jnp.kron(eye8, S).astype(bf)
  b1a8 = jnp.tile(b1a, 8)
  b1b8 = jnp.tile(b1b, 8)
  b2a8 = jnp.tile(b2a, 8)
  b2b8 = jnp.tile(b2b, 8)
  root1_bd = jnp.kron(eye8, root1)
  root2_bd = jnp.kron(eye8, root2)
  Wl_bd = jnp.kron(eye8, Wl)
  bias1_t = jnp.tile(bias1, 8)
  bias2_t = jnp.tile(bias2, 8)
  bl_t = jnp.tile(bl, 8)
  eap = edge_attr.astype(bf).reshape(NCHK, ECH // 8, 8 * EA)

  def layer(table, Wa_bd, ba8, Wb_bd, bb8, with_cnt):
    sums = []
    cnts = []
    for ci in range(NCHK):
      xj = _sc_gather(table, src4[ci])
      msg = _tc_edge_msgs(eap[ci], xj.reshape(ECH // 8, 128),
                          Wa_bd, ba8, Wb_bd, bb8, R_bd, S_bd)
      res = _sc_scatter(msg.reshape(ECH, 16), dst4[ci], with_cnt)
      if with_cnt:
        sums.append(res[0].reshape(NC, NP8, 128))
        cnts.append(res[1].reshape(NC, NP8, 128))
      else:
        sums.append(res.reshape(NC, NP8, 128))
    return sums, cnts

  sums1, cnts1 = layer(x, W1a_bd, b1a8, W1b_bd, b1b8, True)
  h1p, rinvp = _tc_finalize1(sums1, cnts1, x.reshape(NP8, 128),
                             root1_bd, bias1_t)

  sums2, _ = layer(h1p.reshape(N, H), W2a_bd, b2a8, W2b_bd, b2b8, False)
  out = _tc_finalize2(sums2, rinvp, h1p, root2_bd, bias2_t, Wl_bd, bl_t)
  return out.reshape(N)


# speedup vs baseline: 1.1736x; 1.0441x over previous
import functools

import jax
import jax.numpy as jnp
from jax import lax
from jax.experimental import pallas as pl
from jax.experimental.pallas import tpu as pltpu
from jax.experimental.pallas import tpu_sc as plsc

N = 10000
E = 320000
IN = 16
H = 16
EA = 4

NC = 2
NS = 16
NW = NC * NS
NCHK = 1
ECH = E // NCHK
PW = ECH // NW
CH = 80
NCHUNK = PW // CH
GRP = 5
NGRP = NCHUNK // GRP
ZROWS = N // NS


@functools.cache
def _mesh():
  return plsc.VectorSubcoreMesh(core_axis_name="c", subcore_axis_name="s")


_SC_PARAMS = pltpu.CompilerParams(use_tc_tiling_on_sc=False)


def _sc_gather(table, idx3):
  @functools.partial(
      pl.kernel,
      mesh=_mesh(),
      out_type=jax.ShapeDtypeStruct((ECH, 16), jnp.float32),
      scratch_types=[
          pltpu.VMEM((NCHUNK, CH), jnp.int32),
          pltpu.VMEM((2 * GRP, CH, 16), jnp.float32),
          pltpu.SemaphoreType.DMA,
          pltpu.SemaphoreType.DMA,
      ],
      compiler_params=_SC_PARAMS,
  )
  def k(table_hbm, idx_hbm, out_hbm, idx_v, rows_v, gsem, ssem):
    wid = lax.axis_index("s") * NC + lax.axis_index("c")
    base = wid * PW
    pltpu.sync_copy(idx_hbm.at[wid], idx_v)

    def fire(g, p):
      j0 = g * GRP
      for b in range(GRP):
        pltpu.async_copy(
            table_hbm.at[idx_v.at[j0 + b]], rows_v.at[p * GRP + b], gsem)

    def retire(g, p):
      j0 = g * GRP
      for b in range(GRP):
        pltpu.make_async_copy(
            table_hbm.at[idx_v.at[j0 + b]], rows_v.at[p * GRP + b],
            gsem).wait()
      for b in range(GRP):
        pltpu.async_copy(
            rows_v.at[p * GRP + b],
            out_hbm.at[pl.ds(base + (j0 + b) * CH, CH)], ssem)

    def drain_stores(p):
      for b in range(GRP):
        pltpu.make_async_copy(
            rows_v.at[p * GRP + b], out_hbm.at[pl.ds(base, CH)], ssem).wait()

    fire(0, 0)

    def pair(q, carry):
      g = 2 * q

      @pl.when(g + 1 < NGRP)
      def _f1():
        fire(g + 1, 1)

      retire(g, 0)

      @pl.when(g + 2 < NGRP)
      def _f0():
        drain_stores(0)
        fire(g + 2, 0)

      @pl.when(g + 1 < NGRP)
      def _r1():
        retire(g + 1, 1)
        drain_stores(1)

      return carry

    lax.fori_loop(0, (NGRP + 1) // 2, pair, 0)
    drain_stores(0)

  return k(table, idx3)


def _sc_scatter(msg, idx3, with_cnt):
  outs = [jax.ShapeDtypeStruct((NC, N, 16), jnp.float32)]
  scratch = [
      pltpu.VMEM((NCHUNK, CH), jnp.int32),
      pltpu.VMEM((2 * GRP, CH, 16), jnp.float32),
      pltpu.VMEM((ZROWS, 16), jnp.float32),
      pltpu.VMEM_SHARED((N, 16), jnp.float32),
      pltpu.SemaphoreType.DMA,
      pltpu.SemaphoreType.DMA,
      pltpu.SemaphoreType.DMA,
  ]
  if with_cnt:
    outs.append(jax.ShapeDtypeStruct((NC, N, 16), jnp.float32))
    scratch.insert(2, pltpu.VMEM((CH, 16), jnp.float32))
    scratch.insert(4, pltpu.VMEM_SHARED((N, 16), jnp.float32))

  @functools.partial(
      pl.kernel, mesh=_mesh(), out_type=outs, scratch_types=scratch,
      compiler_params=_SC_PARAMS)
  def k(*refs):
    if with_cnt:
      (msg_hbm, idx_hbm, out_sum, out_cnt,
       idx_v, rows_v, ones_v, stage_v, acc, cacc, lsem, asem, csem) = refs
    else:
      (msg_hbm, idx_hbm, out_sum,
       idx_v, rows_v, stage_v, acc, lsem, asem, csem) = refs
    sid = lax.axis_index("s")
    cid = lax.axis_index("c")
    wid = sid * NC + cid
    base = wid * PW

    def zbody(r, carry):
      stage_v[r, :] = jnp.zeros((16,), jnp.float32)
      return carry

    lax.fori_loop(0, ZROWS, zbody, 0)
    pltpu.sync_copy(stage_v, acc.at[pl.ds(sid * ZROWS, ZROWS)])
    if with_cnt:
      pltpu.sync_copy(stage_v, cacc.at[pl.ds(sid * ZROWS, ZROWS)])

      def obody(r, carry):
        ones_v[r, :] = jnp.ones((16,), jnp.float32)
        return carry

      lax.fori_loop(0, CH, obody, 0)
    pltpu.sync_copy(idx_hbm.at[wid], idx_v)
    plsc.subcore_barrier()

    def fire_loads(g, p):
      j0 = g * GRP
      for b in range(GRP):
        pltpu.async_copy(
            msg_hbm.at[pl.ds(base + (j0 + b) * CH, CH)],
            rows_v.at[p * GRP + b], lsem)

    def retire(g, p):
      j0 = g * GRP
      for b in range(GRP):
        pltpu.make_async_copy(
            msg_hbm.at[pl.ds(base, CH)], rows_v.at[p * GRP + b], lsem).wait()
      for b in range(GRP):
        pltpu.async_copy(rows_v.at[p * GRP + b], acc.at[idx_v.at[j0 + b]],
                         asem, add=True)
        if with_cnt:
          pltpu.async_copy(ones_v, cacc.at[idx_v.at[j0 + b]], csem, add=True)

    def drain_adds(p):
      for b in range(GRP):
        pltpu.make_async_copy(
            rows_v.at[p * GRP + b], acc.at[idx_v.at[b]], asem).wait()
        if with_cnt:
          pltpu.make_async_copy(ones_v, cacc.at[idx_v.at[b]], csem).wait()

    fire_loads(0, 0)

    def pair(q, carry):
      g = 2 * q

      @pl.when(g + 1 < NGRP)
      def _f1():
        fire_loads(g + 1, 1)

      retire(g, 0)

      @pl.when(g + 2 < NGRP)
      def _f0():
        drain_adds(0)
        fire_loads(g + 2, 0)

      @pl.when(g + 1 < NGRP)
      def _r1():
        retire(g + 1, 1)
        drain_adds(1)

      return carry

    lax.fori_loop(0, (NGRP + 1) // 2, pair, 0)
    drain_adds(0)
    plsc.subcore_barrier()

    pltpu.sync_copy(acc.at[pl.ds(sid * ZROWS, ZROWS)], stage_v)
    pltpu.sync_copy(stage_v, out_sum.at[cid, pl.ds(sid * ZROWS, ZROWS)])
    if with_cnt:
      pltpu.sync_copy(cacc.at[pl.ds(sid * ZROWS, ZROWS)], stage_v)
      pltpu.sync_copy(stage_v, out_cnt.at[cid, pl.ds(sid * ZROWS, ZROWS)])

  res = k(msg, idx3)
  return res if with_cnt else res[0]


_TB = 6400


def _tc_edge_msgs(eap, xjp, Wa_bd, ba8, Wb_bd, bb8, R_bd, S_bd):
  TB8 = _TB // 8

  def body(eap_ref, xj_ref, wa, ba_r, wb, bb_r, r_r, s_r, out_ref):
    hp = jnp.maximum(
        jnp.dot(eap_ref[...], wa[...], preferred_element_type=jnp.float32)
        + ba_r[...], 0.0)
    hp = hp.astype(jnp.bfloat16)
    wp = jnp.dot(hp, wb[...],
                 preferred_element_type=jnp.float32) + bb_r[...]
    xep = jnp.dot(xj_ref[...].astype(jnp.bfloat16), r_r[...],
                  preferred_element_type=jnp.float32)
    out_ref[...] = jnp.dot(
        (wp * xep).astype(jnp.bfloat16), s_r[...],
        preferred_element_type=jnp.float32)

  zero = lambda i: (0, 0)
  return pl.pallas_call(
      body,
      grid=(ECH // _TB,),
      in_specs=[
          pl.BlockSpec((TB8, 8 * EA), lambda i: (i, 0)),
          pl.BlockSpec((TB8, 128), lambda i: (i, 0)),
          pl.BlockSpec((8 * EA, 8 * 32), zero),
          pl.BlockSpec((1, 8 * 32), zero),
          pl.BlockSpec((8 * 32, 8 * IN * H), zero),
          pl.BlockSpec((1, 8 * IN * H), zero),
          pl.BlockSpec((128, 8 * IN * H), zero),
          pl.BlockSpec((8 * IN * H, 128), zero),
      ],
      out_specs=pl.BlockSpec((TB8, 128), lambda i: (i, 0)),
      out_shape=jax.ShapeDtypeStruct((ECH // 8, 128), jnp.float32),
  )(eap, xjp, Wa_bd, ba8.reshape(1, 8 * 32), Wb_bd,
    bb8.reshape(1, 8 * IN * H), R_bd, S_bd)


NP8 = N // 8


def _tc_finalize1(sums_l, cnts_l, x_p, root_bd, bias_t):
  ns = len(sums_l)

  def body(*refs):
    s_refs = refs[:ns]
    c_refs = refs[ns:2 * ns]
    x_ref, r_ref, b_ref, h_ref, rinv_ref = refs[2 * ns:]
    cnt = sum(c[0] + c[1] for c in c_refs)
    rinv = 1.0 / jnp.maximum(cnt, 1.0)
    mean = sum(s[0] + s[1] for s in s_refs) * rinv
    h = mean + jnp.dot(
        x_ref[...], r_ref[...], preferred_element_type=jnp.float32) + b_ref[...]
    h_ref[...] = jnp.maximum(h, 0.0)
    rinv_ref[...] = rinv

  return pl.pallas_call(
      body,
      out_shape=[
          jax.ShapeDtypeStruct((NP8, 128), jnp.float32),
          jax.ShapeDtypeStruct((NP8, 128), jnp.float32),
      ],
  )(*sums_l, *cnts_l, x_p, root_bd, bias_t.reshape(1, 128))


def _tc_finalize2(sums_l, rinv_p, h1_p, root_bd, bias_t, Wl_bd, bl_t):
  ns = len(sums_l)

  def body(*refs):
    s_refs = refs[:ns]
    rinv_ref, h1_ref, r_ref, b_ref, wl_ref, bl_ref, out_ref = refs[ns:]
    mean = sum(s[0] + s[1] for s in s_refs) * rinv_ref[...]
    h2 = mean + jnp.dot(
        h1_ref[...], r_ref[...], preferred_element_type=jnp.float32) + b_ref[...]
    h2 = jnp.maximum(h2, 0.0)
    out_ref[...] = jnp.dot(
        h2, wl_ref[...], preferred_element_type=jnp.float32) + bl_ref[...]

  return pl.pallas_call(
      body,
      out_shape=jax.ShapeDtypeStruct((NP8, 8), jnp.float32),
  )(*sums_l, rinv_p, h1_p, root_bd, bias_t.reshape(1, 128), Wl_bd,
    bl_t.reshape(1, 8))


def kernel(x, edge_index, edge_attr, W1a, b1a, W1b, b1b, root1, bias1,
           W2a, b2a, W2b, b2b, root2, bias2, Wl, bl):
  src4 = edge_index[0].astype(jnp.int32).reshape(NCHK, NW, NCHUNK, CH)
  dst4 = edge_index[1].astype(jnp.int32).reshape(NCHK, NW, NCHUNK, CH)

  c = jnp.arange(IN * H, dtype=jnp.int32)
  R = (jnp.arange(IN, dtype=jnp.int32)[:, None] == (c // H)[None, :]
       ).astype(jnp.float32)
  S = ((c % H)[:, None] == jnp.arange(H, dtype=jnp.int32)[None, :]
       ).astype(jnp.float32)

  eye8 = jnp.eye(8, dtype=jnp.float32)
  bf = jnp.bfloat16
  W1a_bd = jnp.kron(eye8, W1a).astype(bf)
  W1b_bd = jnp.kron(eye8, W1b).astype(bf)
  W2a_bd = jnp.kron(eye8, W2a).astype(bf)
  W2b_bd = jnp.kron(eye8, W2b).astype(bf)
  R_bd = jnp.kron(eye8, R).astype(bf)
  S_bd = jnp.kron(eye8, S).astype(bf)
  b1a8 = jnp.tile(b1a, 8)
  b1b8 = jnp.tile(b1b, 8)
  b2a8 = jnp.tile(b2a, 8)
  b2b8 = jnp.tile(b2b, 8)
  root1_bd = jnp.kron(eye8, root1)
  root2_bd = jnp.kron(eye8, root2)
  Wl_bd = jnp.kron(eye8, Wl)
  bias1_t = jnp.tile(bias1, 8)
  bias2_t = jnp.tile(bias2, 8)
  bl_t = jnp.tile(bl, 8)
  eap = edge_attr.astype(bf).reshape(NCHK, ECH // 8, 8 * EA)

  def layer(table, Wa_bd, ba8, Wb_bd, bb8, with_cnt):
    sums = []
    cnts = []
    for ci in range(NCHK):
      xj = _sc_gather(table, src4[ci])
      msg = _tc_edge_msgs(eap[ci], xj.reshape(ECH // 8, 128),
                          Wa_bd, ba8, Wb_bd, bb8, R_bd, S_bd)
      res = _sc_scatter(msg.reshape(ECH, 16), dst4[ci], with_cnt)
      if with_cnt:
        sums.append(res[0].reshape(NC, NP8, 128))
        cnts.append(res[1].reshape(NC, NP8, 128))
      else:
        sums.append(res.reshape(NC, NP8, 128))
    return sums, cnts

  sums1, cnts1 = layer(x, W1a_bd, b1a8, W1b_bd, b1b8, True)
  h1p, rinvp = _tc_finalize1(sums1, cnts1, x.reshape(NP8, 128),
                             root1_bd, bias1_t)

  sums2, _ = layer(h1p.reshape(N, H), W2a_bd, b2a8, W2b_bd, b2b8, False)
  out = _tc_finalize2(sums2, rinvp, h1p, root2_bd, bias2_t, Wl_bd, bl_t)
  return out.reshape(N)
